# Initial kernel scaffold; baseline (speedup 1.0000x reference)
import functools

import jax
import jax.numpy as jnp
from jax import lax
from jax.experimental import pallas as pl
from jax.experimental.pallas import tpu as pltpu
from jax.experimental.pallas import tpu_sc as plsc

VOCAB = 1000000
D = 64
BATCH = 16384
HIST = 50

NC = 2
NS = 16
NW = NC * NS

GRAN = 2 * HIST
TOTAL = BATCH * HIST
ROWS = TOTAL // GRAN
ROWS_PER_W = ROWS // NW
G_PER_CHUNK = 4
CHUNK = G_PER_CHUNK * GRAN
NCHUNK = ROWS_PER_W // G_PER_CHUNK


def _sc_partials(x2, table, hpat, zeros):
  mesh = plsc.VectorSubcoreMesh(core_axis_name="c", subcore_axis_name="s")

  @functools.partial(
      pl.kernel,
      out_type=jax.ShapeDtypeStruct((NW, HIST, D), jnp.float32),
      mesh=mesh,
      scratch_types=[
          pltpu.VMEM((ROWS_PER_W, GRAN), jnp.int32),
          pltpu.VMEM((CHUNK, D), jnp.float32),
          pltpu.VMEM((CHUNK, D), jnp.float32),
          pltpu.VMEM((1, GRAN), jnp.int32),
          pltpu.VMEM_SHARED((NS * HIST, D), jnp.float32),
          pltpu.SemaphoreType.DMA,
          pltpu.SemaphoreType.DMA,
      ],
  )
  def k(x_hbm, tab_hbm, hpat_hbm, zeros_hbm, out_hbm,
        idx_v, rows_a, rows_b, hpat_v, acc_sh, sem_a, sem_b):
    cid = lax.axis_index("c")
    sid = lax.axis_index("s")
    wid = sid * NC + cid
    row_base = wid * ROWS_PER_W

    pltpu.sync_copy(hpat_hbm.at[pl.ds(sid, 1)], hpat_v)
    pltpu.sync_copy(zeros_hbm, acc_sh.at[pl.ds(sid * HIST, HIST)])
    pltpu.sync_copy(x_hbm.at[pl.ds(row_base, ROWS_PER_W)], idx_v)

    def fire(chunk, rbuf, sem):
      for g in range(G_PER_CHUNK):
        pltpu.make_async_copy(
            tab_hbm.at[idx_v.at[chunk * G_PER_CHUNK + g]],
            rbuf.at[pl.ds(g * GRAN, GRAN)],
            sem,
        ).start()

    def drain(chunk, rbuf, sem):
      for g in range(G_PER_CHUNK):
        pltpu.make_async_copy(
            tab_hbm.at[idx_v.at[chunk * G_PER_CHUNK + g]],
            rbuf.at[pl.ds(g * GRAN, GRAN)],
            sem,
        ).wait()

    def scatter_add(rbuf):
      for g in range(G_PER_CHUNK):
        pltpu.sync_copy(
            rbuf.at[pl.ds(g * GRAN, GRAN)],
            acc_sh.at[hpat_v.at[0]],
            add=True,
        )

    fire(0, rows_a, sem_a)

    @pl.loop(0, NCHUNK, step=2)
    def _(c):
      fire(c + 1, rows_b, sem_b)
      drain(c, rows_a, sem_a)
      scatter_add(rows_a)

      @pl.when(c + 2 < NCHUNK)
      def _():
        fire(c + 2, rows_a, sem_a)

      drain(c + 1, rows_b, sem_b)
      scatter_add(rows_b)

    pltpu.sync_copy(acc_sh.at[pl.ds(sid * HIST, HIST)], out_hbm.at[wid])

  return k(x2, table, hpat, zeros)


def _combine(partials):
  def body(p_ref, o_ref):
    o_ref[...] = jnp.sum(p_ref[...], axis=0) * (1.0 / BATCH)

  return pl.pallas_call(
      body,
      out_shape=jax.ShapeDtypeStruct((HIST, D), jnp.float32),
  )(partials)


@jax.jit
def kernel(x, table):
  x2 = x.reshape(ROWS, GRAN).astype(jnp.int32)
  hpat = (jnp.arange(NS, dtype=jnp.int32)[:, None] * HIST
          + (jnp.arange(GRAN, dtype=jnp.int32) % HIST)[None, :])
  zeros = jnp.zeros((HIST, D), jnp.float32)
  partials = _sc_partials(x2, table, hpat, zeros)
  return _combine(partials)


# speedup vs baseline: 2.7618x; 2.7618x over previous
import functools

import jax
import jax.numpy as jnp
from jax import lax
from jax.experimental import pallas as pl
from jax.experimental.pallas import tpu as pltpu
from jax.experimental.pallas import tpu_sc as plsc

VOCAB = 1000000
D = 64
BATCH = 16384
HIST = 50

NC = 2
NS = 16
NW = NC * NS

GRAN = 2 * HIST
TOTAL = BATCH * HIST
ROWS = TOTAL // GRAN
ROWS_PER_W = ROWS // NW
G_PER_CHUNK = 4
CHUNK = G_PER_CHUNK * GRAN
NCHUNK = ROWS_PER_W // G_PER_CHUNK


def _sc_partials(x2, table, hpat, zeros):
  mesh = plsc.VectorSubcoreMesh(core_axis_name="c", subcore_axis_name="s")

  @functools.partial(
      pl.kernel,
      out_type=jax.ShapeDtypeStruct((NW, HIST, D), jnp.float32),
      mesh=mesh,
      compiler_params=pltpu.CompilerParams(use_tc_tiling_on_sc=False),
      scratch_types=[
          pltpu.VMEM((ROWS_PER_W, GRAN), jnp.int32),
          pltpu.VMEM((CHUNK, D), jnp.float32),
          pltpu.VMEM((CHUNK, D), jnp.float32),
          pltpu.VMEM((1, GRAN), jnp.int32),
          pltpu.VMEM_SHARED((NS * HIST, D), jnp.float32),
          pltpu.SemaphoreType.DMA,
          pltpu.SemaphoreType.DMA,
      ],
  )
  def k(x_hbm, tab_hbm, hpat_hbm, zeros_hbm, out_hbm,
        idx_v, rows_a, rows_b, hpat_v, acc_sh, sem_a, sem_b):
    cid = lax.axis_index("c")
    sid = lax.axis_index("s")
    wid = sid * NC + cid
    row_base = wid * ROWS_PER_W

    pltpu.sync_copy(hpat_hbm.at[pl.ds(sid, 1)], hpat_v)
    pltpu.sync_copy(zeros_hbm, acc_sh.at[pl.ds(sid * HIST, HIST)])
    pltpu.sync_copy(x_hbm.at[pl.ds(row_base, ROWS_PER_W)], idx_v)

    def fire(chunk, rbuf, sem):
      for g in range(G_PER_CHUNK):
        pltpu.make_async_copy(
            tab_hbm.at[idx_v.at[chunk * G_PER_CHUNK + g]],
            rbuf.at[pl.ds(g * GRAN, GRAN)],
            sem,
        ).start()

    def drain(chunk, rbuf, sem):
      for g in range(G_PER_CHUNK):
        pltpu.make_async_copy(
            tab_hbm.at[idx_v.at[chunk * G_PER_CHUNK + g]],
            rbuf.at[pl.ds(g * GRAN, GRAN)],
            sem,
        ).wait()

    def scatter_add(rbuf):
      for g in range(G_PER_CHUNK):
        pltpu.sync_copy(
            rbuf.at[pl.ds(g * GRAN, GRAN)],
            acc_sh.at[hpat_v.at[0]],
            add=True,
        )

    fire(0, rows_a, sem_a)

    @pl.loop(0, NCHUNK, step=2)
    def _(c):
      fire(c + 1, rows_b, sem_b)
      drain(c, rows_a, sem_a)
      scatter_add(rows_a)

      @pl.when(c + 2 < NCHUNK)
      def _():
        fire(c + 2, rows_a, sem_a)

      drain(c + 1, rows_b, sem_b)
      scatter_add(rows_b)

    pltpu.sync_copy(acc_sh.at[pl.ds(sid * HIST, HIST)], out_hbm.at[wid])

  return k(x2, table, hpat, zeros)


def _combine(partials):
  def body(p_ref, o_ref):
    o_ref[...] = jnp.sum(p_ref[...], axis=0) * (1.0 / BATCH)

  return pl.pallas_call(
      body,
      out_shape=jax.ShapeDtypeStruct((HIST, D), jnp.float32),
  )(partials)


@jax.jit
def kernel(x, table):
  x2 = x.reshape(ROWS, GRAN).astype(jnp.int32)
  hpat = (jnp.arange(NS, dtype=jnp.int32)[:, None] * HIST
          + (jnp.arange(GRAN, dtype=jnp.int32) % HIST)[None, :])
  zeros = jnp.zeros((HIST, D), jnp.float32)
  partials = _sc_partials(x2, table, hpat, zeros)
  return _combine(partials)


# speedup vs baseline: 2.7698x; 1.0029x over previous
import functools

import jax
import jax.numpy as jnp
from jax import lax
from jax.experimental import pallas as pl
from jax.experimental.pallas import tpu as pltpu
from jax.experimental.pallas import tpu_sc as plsc

VOCAB = 1000000
D = 64
BATCH = 16384
HIST = 50

NC = 2
NS = 16
NW = NC * NS

GRAN = 2 * HIST
TOTAL = BATCH * HIST
ROWS = TOTAL // GRAN
ROWS_PER_W = ROWS // NW
G_PER_CHUNK = 2
CHUNK = G_PER_CHUNK * GRAN
NCHUNK = ROWS_PER_W // G_PER_CHUNK


def _sc_partials(x2, table, hpat, zeros):
  mesh = plsc.VectorSubcoreMesh(core_axis_name="c", subcore_axis_name="s")

  @functools.partial(
      pl.kernel,
      out_type=jax.ShapeDtypeStruct((NW, HIST, D), jnp.float32),
      mesh=mesh,
      compiler_params=pltpu.CompilerParams(use_tc_tiling_on_sc=False),
      scratch_types=[
          pltpu.VMEM((ROWS_PER_W, GRAN), jnp.int32),
          pltpu.VMEM((CHUNK, D), jnp.float32),
          pltpu.VMEM((CHUNK, D), jnp.float32),
          pltpu.VMEM((CHUNK, D), jnp.float32),
          pltpu.VMEM((CHUNK, D), jnp.float32),
          pltpu.VMEM((1, GRAN), jnp.int32),
          pltpu.VMEM_SHARED((NS * HIST, D), jnp.float32),
          pltpu.SemaphoreType.DMA,
          pltpu.SemaphoreType.DMA,
          pltpu.SemaphoreType.DMA,
          pltpu.SemaphoreType.DMA,
          pltpu.SemaphoreType.DMA,
          pltpu.SemaphoreType.DMA,
          pltpu.SemaphoreType.DMA,
          pltpu.SemaphoreType.DMA,
      ],
  )
  def k(x_hbm, tab_hbm, hpat_hbm, zeros_hbm, out_hbm,
        idx_v, r0, r1, r2, r3, hpat_v, acc_sh,
        g0, g1, g2, g3, s0, s1, s2, s3):
    cid = lax.axis_index("c")
    sid = lax.axis_index("s")
    wid = sid * NC + cid
    row_base = wid * ROWS_PER_W
    rbufs = [r0, r1, r2, r3]
    gsems = [g0, g1, g2, g3]
    ssems = [s0, s1, s2, s3]

    pltpu.sync_copy(hpat_hbm.at[pl.ds(sid, 1)], hpat_v)
    pltpu.sync_copy(zeros_hbm, acc_sh.at[pl.ds(sid * HIST, HIST)])
    pltpu.sync_copy(x_hbm.at[pl.ds(row_base, ROWS_PER_W)], idx_v)

    def gather_desc(chunk, b, g):
      return pltpu.make_async_copy(
          tab_hbm.at[idx_v.at[chunk * G_PER_CHUNK + g]],
          rbufs[b].at[pl.ds(g * GRAN, GRAN)],
          gsems[b],
      )

    def fire_g(chunk, b):
      for g in range(G_PER_CHUNK):
        gather_desc(chunk, b, g).start()

    def drain_g(chunk, b):
      for g in range(G_PER_CHUNK):
        gather_desc(chunk, b, g).wait()

    def fire_s(b):
      for g in range(G_PER_CHUNK):
        pltpu.async_copy(
            rbufs[b].at[pl.ds(g * GRAN, GRAN)],
            acc_sh.at[hpat_v.at[0]],
            ssems[b],
            add=True,
        )

    def drain_s(b):
      for g in range(G_PER_CHUNK):
        pltpu.make_async_copy(
            rbufs[b].at[pl.ds(g * GRAN, GRAN)],
            acc_sh.at[hpat_v.at[0]],
            ssems[b],
        ).wait()

    fire_g(0, 0)
    fire_g(1, 1)

    @pl.loop(0, NCHUNK, step=4)
    def _(c):
      for p in range(4):
        k_chunk = c + p

        @pl.when(k_chunk >= 2)
        def _():
          drain_s((p + 2) % 4)

        @pl.when(k_chunk + 2 < NCHUNK)
        def _():
          fire_g(k_chunk + 2, (p + 2) % 4)

        drain_g(k_chunk, p)
        fire_s(p)

    drain_s(2)
    drain_s(3)
    pltpu.sync_copy(acc_sh.at[pl.ds(sid * HIST, HIST)], out_hbm.at[wid])

  return k(x2, table, hpat, zeros)


def _combine(partials):
  def body(p_ref, o_ref):
    o_ref[...] = jnp.sum(p_ref[...], axis=0) * (1.0 / BATCH)

  return pl.pallas_call(
      body,
      out_shape=jax.ShapeDtypeStruct((HIST, D), jnp.float32),
  )(partials)


@jax.jit
def kernel(x, table):
  x2 = x.reshape(ROWS, GRAN).astype(jnp.int32)
  hpat = (jnp.arange(NS, dtype=jnp.int32)[:, None] * HIST
          + (jnp.arange(GRAN, dtype=jnp.int32) % HIST)[None, :])
  zeros = jnp.zeros((HIST, D), jnp.float32)
  partials = _sc_partials(x2, table, hpat, zeros)
  return _combine(partials)
